# Initial kernel scaffold; baseline (speedup 1.0000x reference)
import jax
import jax.numpy as jnp
from jax.experimental import pallas as pl


def kernel(x, edge_index, batch, params):
    raise NotImplementedError("write your pallas kernel here")


# speedup vs baseline: 3.1871x; 3.1871x over previous
import functools
import math

import jax
import jax.numpy as jnp
import numpy as np
from jax import lax
from jax.experimental import pallas as pl
from jax.experimental.pallas import tpu as pltpu
from jax.experimental.pallas import tpu_sc as plsc

N = 10000
E = 320000
D = 128
H = 100
T = 5
G = 64
NL = 3
DEG_HIST = np.zeros(33, dtype=np.float64)
DEG_HIST[32] = N
_bins = np.arange(33, dtype=np.float64)
AVG_LOG_CONST = float((np.log(_bins + 1.0) * DEG_HIST).sum() / DEG_HIST.sum())

NW = 32
NBKT = 64
BKT = 160
NP = NBKT * BKT
CAP = 8192
NB = CAP // 128
CH = 8000
NCHUNK = 1 + T

_mesh = plsc.VectorSubcoreMesh(core_axis_name="c", subcore_axis_name="s")


@functools.partial(
    pl.kernel,
    out_type=(
        jax.ShapeDtypeStruct((NBKT, NB, 128), jnp.int32),
        jax.ShapeDtypeStruct((NBKT, CAP), jnp.int32),
        jax.ShapeDtypeStruct((NBKT, 16), jnp.int32),
        jax.ShapeDtypeStruct((NW, 328, 16), jnp.float32),
    ),
    mesh=_mesh,
    scratch_types=[
        pltpu.VMEM((CH,), jnp.int32),
        pltpu.VMEM((CH,), jnp.int32),
        pltpu.VMEM((CAP,), jnp.int32),
        pltpu.VMEM((CAP,), jnp.int32),
        pltpu.VMEM((CAP,), jnp.int32),
        pltpu.VMEM((CAP,), jnp.int32),
        pltpu.VMEM((328, 16), jnp.float32),
        pltpu.VMEM((16,), jnp.int32),
        pltpu.SemaphoreType.DMA,
    ],
)
def _filter_kernel(src_hbm, dst_hbm, wsrc_hbm, wdst_hbm, cnt_hbm, deg_hbm,
                   sv, dv, ws0, wd0, ws1, wd1, degv, cntv, sem):
    w = lax.axis_index("s") * 2 + lax.axis_index("c")
    lo = w * 320

    def zdeg(i, _):
        degv[i, :] = jnp.zeros((16,), jnp.float32)
        return 0
    lax.fori_loop(0, 328, zdeg, 0)

    def zwl(i, _):
        z = jnp.zeros((16,), jnp.int32)
        d = jnp.full((16,), BKT, jnp.int32)
        ws0[pl.ds(i * 16, 16)] = z
        wd0[pl.ds(i * 16, 16)] = d
        ws1[pl.ds(i * 16, 16)] = z
        wd1[pl.ds(i * 16, 16)] = d
        return 0
    lax.fori_loop(0, CAP // 16, zwl, 0)

    def chunk(cix, carry):
        c0, c1 = carry
        pltpu.sync_copy(src_hbm.at[pl.ds(cix * CH, CH)], sv)
        pltpu.sync_copy(dst_hbm.at[pl.ds(cix * CH, CH)], dv)

        def vec(i, carry):
            c0, c1 = carry
            d16 = dv[pl.ds(i * 16, 16)]
            s16 = sv[pl.ds(i * 16, 16)]
            for j in range(16):
                dj = d16[j]
                dl = dj - lo
                ok = (dl >= 0) & (dl < 320)
                ok0 = ok & (dl < BKT) & (c0 < CAP - 16)
                ok1 = ok & (dl >= BKT) & (c1 < CAP - 16)

                @pl.when(ok0)
                def _():
                    ws0[pl.ds(c0, 16)] = jnp.broadcast_to(s16[j], (16,))
                    wd0[pl.ds(c0, 16)] = jnp.broadcast_to(dl, (16,))
                    degv[dl, :] = degv[dl, :] + 1.0

                @pl.when(ok1)
                def _():
                    ws1[pl.ds(c1, 16)] = jnp.broadcast_to(s16[j], (16,))
                    wd1[pl.ds(c1, 16)] = jnp.broadcast_to(dl - BKT, (16,))
                    degv[dl, :] = degv[dl, :] + 1.0

                c0 = jnp.where(ok0, c0 + 1, c0)
                c1 = jnp.where(ok1, c1 + 1, c1)
            return (c0, c1)
        return lax.fori_loop(0, CH // 16, vec, (c0, c1))

    c0, c1 = lax.fori_loop(0, E // CH, chunk, (0, 0))

    for p, (wsv, wdv, cnt) in enumerate([(ws0, wd0, c0), (ws1, wd1, c1)]):
        bkt = 2 * w + p
        for b in range(NB):
            pltpu.sync_copy(wsv.at[pl.ds(b * 128, 128)], wsrc_hbm.at[bkt].at[b])
        pltpu.sync_copy(wdv, wdst_hbm.at[bkt])
        cntv[pl.ds(0, 16)] = jnp.broadcast_to(cnt, (16,))
        pltpu.sync_copy(cntv, cnt_hbm.at[bkt])
    pltpu.sync_copy(degv, deg_hbm.at[w])


@functools.partial(
    pl.kernel,
    out_type=(
        jax.ShapeDtypeStruct((NP, 128), jnp.float32),
        jax.ShapeDtypeStruct((T, NP, 128), jnp.float32),
        jax.ShapeDtypeStruct((T, NP, 128), jnp.float32),
        jax.ShapeDtypeStruct((T, NP, 128), jnp.float32),
    ),
    mesh=_mesh,
    scratch_types=[
        pltpu.VMEM((NB, 128), jnp.int32),
        pltpu.VMEM((CAP,), jnp.int32),
        pltpu.VMEM((16,), jnp.int32),
        pltpu.VMEM((2, 128, 128), jnp.float32),
        pltpu.VMEM((168, 128), jnp.float32),
        pltpu.VMEM((168, 128), jnp.float32),
        pltpu.VMEM((168, 128), jnp.float32),
        pltpu.SemaphoreType.DMA,
    ],
)
def _segment_kernel(tab_hbm, wsrc_hbm, wdst_hbm, cnt_hbm,
                    nsx_hbm, ssq_hbm, mn_hbm, mx_hbm,
                    wsv, wdv, cntv, rows, asum, amin, amax, sem):
    w = lax.axis_index("s") * 2 + lax.axis_index("c")
    zero = jnp.zeros((16,), jnp.float32)
    big = jnp.full((16,), 3e38, jnp.float32)

    def pass_body(p, _):
        bkt = 2 * w + p
        base = pl.multiple_of(bkt * BKT, 8)
        pltpu.sync_copy(wsrc_hbm.at[bkt], wsv)
        pltpu.sync_copy(wdst_hbm.at[bkt], wdv)
        pltpu.sync_copy(cnt_hbm.at[bkt], cntv)
        cnt = cntv[pl.ds(0, 16)][0]
        nb = (cnt + 127) // 128

        def zsum(i, _):
            for q in range(8):
                asum[i, pl.ds(q * 16, 16)] = zero
            return 0
        lax.fori_loop(0, 168, zsum, 0)

        def batch0(b, _):
            buf = b % 2
            pltpu.async_copy(tab_hbm.at[0].at[wsv.at[b]],
                             rows.at[buf], sem).wait()

            def grp(g, _):
                d16 = wdv[pl.ds(b * 128 + g * 16, 16)]
                for j in range(16):
                    dl = d16[j]
                    e = g * 16 + j
                    for q in range(8):
                        sl = pl.ds(q * 16, 16)
                        asum[dl, sl] = asum[dl, sl] + rows[buf, e, sl]
                return 0
            lax.fori_loop(0, 8, grp, 0)
            return 0
        lax.fori_loop(0, nb, batch0, 0)
        pltpu.sync_copy(asum.at[pl.ds(0, BKT)], nsx_hbm.at[pl.ds(base, BKT)])

        def tower(ch, _):
            def zacc(i, _):
                for q in range(8):
                    asum[i, pl.ds(q * 16, 16)] = zero
                    amin[i, pl.ds(q * 16, 16)] = big
                    amax[i, pl.ds(q * 16, 16)] = -big
                return 0
            lax.fori_loop(0, 168, zacc, 0)

            def batch(b, _):
                buf = b % 2
                pltpu.async_copy(tab_hbm.at[ch].at[wsv.at[b]],
                                 rows.at[buf], sem).wait()

                def grp(g, _):
                    d16 = wdv[pl.ds(b * 128 + g * 16, 16)]
                    for j in range(16):
                        dl = d16[j]
                        e = g * 16 + j
                        for q in range(8):
                            sl = pl.ds(q * 16, 16)
                            v = rows[buf, e, sl]
                            amin[dl, sl] = jnp.minimum(amin[dl, sl], v)
                            amax[dl, sl] = jnp.maximum(amax[dl, sl], v)
                            asum[dl, sl] = asum[dl, sl] + v * v
                    return 0
                lax.fori_loop(0, 8, grp, 0)
                return 0
            lax.fori_loop(0, nb, batch, 0)

            t = ch - 1
            pltpu.sync_copy(asum.at[pl.ds(0, BKT)],
                            ssq_hbm.at[t].at[pl.ds(base, BKT)])
            pltpu.sync_copy(amin.at[pl.ds(0, BKT)],
                            mn_hbm.at[t].at[pl.ds(base, BKT)])
            pltpu.sync_copy(amax.at[pl.ds(0, BKT)],
                            mx_hbm.at[t].at[pl.ds(base, BKT)])
            return 0
        lax.fori_loop(1, NCHUNK, tower, 0)
        return 0
    lax.fori_loop(0, 2, pass_body, 0)


def _pad128(a):
    n, f = a.shape
    if f == 128:
        return a
    return jnp.pad(a, ((0, 0), (0, 128 - f)))


def _pna_layer(h, p, wsrc, wdst, cnts, deg, avg_log):
    n, Fi = h.shape
    W1 = p['Wpre'][:, :Fi, :]
    W2 = p['Wpre'][:, Fi:, :]
    Bt = jnp.einsum('nf,tfg->tng', h, W2)
    tabs = jnp.concatenate([_pad128(h)[None],
                            jax.vmap(_pad128)(Bt)], axis=0)
    nsx, ssq, mnb, mxb = _segment_kernel(tabs, wsrc, wdst, cnts)
    NSx = nsx[:n, :Fi]
    degc = jnp.clip(deg, 1.0, None)
    logd = jnp.log(degc + 1.0)[:, None]
    r1 = logd / avg_log
    r2 = avg_log / logd
    dcol = deg[:, None]
    has = dcol > 0
    outs = []
    for t in range(T):
        c = h @ W1[t] + p['bpre'][t]
        sumB = NSx @ W2[t]
        s = dcol * c + sumB
        mean = s / degc[:, None]
        s2 = dcol * c * c + 2.0 * c * sumB + ssq[t, :n, :Fi]
        var = jnp.maximum(s2 / degc[:, None] - mean * mean, 0.0)
        std = jnp.sqrt(var + 1e-5)
        mn = jnp.where(has, c + mnb[t, :n, :Fi], 0.0)
        mx = jnp.where(has, c + mxb[t, :n, :Fi], 0.0)
        agg = jnp.concatenate([mean, mn, mx, std], axis=-1)
        out = jnp.concatenate([h, agg, agg * r1, agg * r2], axis=-1)
        outs.append(out @ p['Wpost'][t] + p['bpost'][t])
    o = jnp.concatenate(outs, axis=-1)
    o = o @ p['Wlin'] + p['blin']
    m = jnp.mean(o, axis=0)
    v = jnp.var(o, axis=0)
    o = p['gamma'] * (o - m) / jnp.sqrt(v + 1e-5) + p['beta']
    return jax.nn.relu(o)


def kernel(x, edge_index, batch, params):
    src = edge_index[0]
    dst = edge_index[1]
    wsrc, wdst, cnts, degr = _filter_kernel(src, dst)
    deg = degr[:, :320, 0].reshape(NP)[:N]

    h = x
    for l in range(NL):
        h = _pna_layer(h, params['conv%d' % l], wsrc, wdst, cnts, deg,
                       AVG_LOG_CONST)

    onehot = (batch[:, None] == jnp.arange(G)[None, :]).astype(h.dtype)
    s = onehot.T @ h
    c = jnp.sum(onehot, axis=0)
    g = s / jnp.clip(c, 1.0, None)[:, None]
    m = params['mlp']
    g = jax.nn.relu(g @ m['W1'] + m['b1'])
    g = jax.nn.relu(g @ m['W2'] + m['b2'])
    return g @ m['W3'] + m['b3']


# speedup vs baseline: 3.6140x; 1.1340x over previous
import functools
import math

import jax
import jax.numpy as jnp
import numpy as np
from jax import lax
from jax.experimental import pallas as pl
from jax.experimental.pallas import tpu as pltpu
from jax.experimental.pallas import tpu_sc as plsc

N = 10000
E = 320000
D = 128
H = 100
T = 5
G = 64
NL = 3
DEG_HIST = np.zeros(33, dtype=np.float64)
DEG_HIST[32] = N
_bins = np.arange(33, dtype=np.float64)
AVG_LOG_CONST = float((np.log(_bins + 1.0) * DEG_HIST).sum() / DEG_HIST.sum())

NW = 32
NBKT = 64
BKT = 160
NP = NBKT * BKT
CAP = 8192
NB = CAP // 128
CH = 8000
NCHUNK = 1 + T

_mesh = plsc.VectorSubcoreMesh(core_axis_name="c", subcore_axis_name="s")


@functools.partial(
    pl.kernel,
    out_type=(
        jax.ShapeDtypeStruct((NBKT, NB, 128), jnp.int32),
        jax.ShapeDtypeStruct((NBKT, CAP), jnp.int32),
        jax.ShapeDtypeStruct((NBKT, 16), jnp.int32),
        jax.ShapeDtypeStruct((NW, 328, 16), jnp.float32),
    ),
    mesh=_mesh,
    scratch_types=[
        pltpu.VMEM((CH,), jnp.int32),
        pltpu.VMEM((CH,), jnp.int32),
        pltpu.VMEM((CAP,), jnp.int32),
        pltpu.VMEM((CAP,), jnp.int32),
        pltpu.VMEM((CAP,), jnp.int32),
        pltpu.VMEM((CAP,), jnp.int32),
        pltpu.VMEM((328, 16), jnp.float32),
        pltpu.VMEM((16,), jnp.int32),
        pltpu.SemaphoreType.DMA,
    ],
)
def _filter_kernel(src_hbm, dst_hbm, wsrc_hbm, wdst_hbm, cnt_hbm, deg_hbm,
                   sv, dv, ws0, wd0, ws1, wd1, degv, cntv, sem):
    w = lax.axis_index("s") * 2 + lax.axis_index("c")
    lo = w * 320

    def zdeg(i, _):
        degv[i, :] = jnp.zeros((16,), jnp.float32)
        return 0
    lax.fori_loop(0, 328, zdeg, 0)

    def zwl(i, _):
        z = jnp.zeros((16,), jnp.int32)
        d = jnp.full((16,), BKT, jnp.int32)
        ws0[pl.ds(i * 16, 16)] = z
        wd0[pl.ds(i * 16, 16)] = d
        ws1[pl.ds(i * 16, 16)] = z
        wd1[pl.ds(i * 16, 16)] = d
        return 0
    lax.fori_loop(0, CAP // 16, zwl, 0)

    def chunk(cix, carry):
        c0, c1 = carry
        pltpu.sync_copy(src_hbm.at[pl.ds(cix * CH, CH)], sv)
        pltpu.sync_copy(dst_hbm.at[pl.ds(cix * CH, CH)], dv)

        def vec(i, carry):
            c0, c1 = carry
            d16 = dv[pl.ds(i * 16, 16)]
            s16 = sv[pl.ds(i * 16, 16)]
            for j in range(16):
                dj = d16[j]
                dl = dj - lo
                ok = (dl >= 0) & (dl < 320)
                ok0 = ok & (dl < BKT) & (c0 < CAP - 16)
                ok1 = ok & (dl >= BKT) & (c1 < CAP - 16)

                @pl.when(ok0)
                def _():
                    ws0[pl.ds(c0, 16)] = jnp.broadcast_to(s16[j], (16,))
                    wd0[pl.ds(c0, 16)] = jnp.broadcast_to(dl, (16,))
                    degv[dl, :] = degv[dl, :] + 1.0

                @pl.when(ok1)
                def _():
                    ws1[pl.ds(c1, 16)] = jnp.broadcast_to(s16[j], (16,))
                    wd1[pl.ds(c1, 16)] = jnp.broadcast_to(dl - BKT, (16,))
                    degv[dl, :] = degv[dl, :] + 1.0

                c0 = jnp.where(ok0, c0 + 1, c0)
                c1 = jnp.where(ok1, c1 + 1, c1)
            return (c0, c1)
        return lax.fori_loop(0, CH // 16, vec, (c0, c1))

    c0, c1 = lax.fori_loop(0, E // CH, chunk, (0, 0))

    for p, (wsv, wdv, cnt) in enumerate([(ws0, wd0, c0), (ws1, wd1, c1)]):
        bkt = 2 * w + p
        for b in range(NB):
            pltpu.sync_copy(wsv.at[pl.ds(b * 128, 128)], wsrc_hbm.at[bkt].at[b])
        pltpu.sync_copy(wdv, wdst_hbm.at[bkt])
        cntv[pl.ds(0, 16)] = jnp.broadcast_to(cnt, (16,))
        pltpu.sync_copy(cntv, cnt_hbm.at[bkt])
    pltpu.sync_copy(degv, deg_hbm.at[w])


@functools.partial(
    pl.kernel,
    out_type=(
        jax.ShapeDtypeStruct((NP, 128), jnp.float32),
        jax.ShapeDtypeStruct((T, NP, 128), jnp.float32),
        jax.ShapeDtypeStruct((T, NP, 128), jnp.float32),
        jax.ShapeDtypeStruct((T, NP, 128), jnp.float32),
    ),
    mesh=_mesh,
    scratch_types=[
        pltpu.VMEM((NB, 128), jnp.int32),
        pltpu.VMEM((CAP,), jnp.int32),
        pltpu.VMEM((16,), jnp.int32),
        pltpu.VMEM((2, 128, 128), jnp.float32),
        pltpu.VMEM((168, 128), jnp.float32),
        pltpu.VMEM((168, 128), jnp.float32),
        pltpu.VMEM((168, 128), jnp.float32),
        pltpu.SemaphoreType.DMA((2,)),
    ],
)
def _segment_kernel(tab_hbm, wsrc_hbm, wdst_hbm, cnt_hbm,
                    nsx_hbm, ssq_hbm, mn_hbm, mx_hbm,
                    wsv, wdv, cntv, rows, asum, amin, amax, sem):
    w = lax.axis_index("s") * 2 + lax.axis_index("c")
    zero = jnp.zeros((16,), jnp.float32)
    big = jnp.full((16,), 3e38, jnp.float32)

    def pass_body(p, _):
        bkt = 2 * w + p
        base = pl.multiple_of(bkt * BKT, 8)
        pltpu.sync_copy(wsrc_hbm.at[bkt], wsv)
        pltpu.sync_copy(wdst_hbm.at[bkt], wdv)
        pltpu.sync_copy(cnt_hbm.at[bkt], cntv)
        cnt = cntv[pl.ds(0, 16)][0]
        nb = (cnt + 127) // 128

        def zsum(i, _):
            for q in range(8):
                asum[i, pl.ds(q * 16, 16)] = zero
            return 0
        lax.fori_loop(0, 168, zsum, 0)

        @pl.when(nb > 0)
        def _():
            pltpu.async_copy(tab_hbm.at[0].at[wsv.at[0]], rows.at[0],
                             sem.at[0])

        def batch0(b, _):
            buf = b % 2

            @pl.when(b + 1 < nb)
            def _():
                nxt = (b + 1) % 2
                pltpu.async_copy(tab_hbm.at[0].at[wsv.at[b + 1]],
                                 rows.at[nxt], sem.at[nxt])
            pltpu.make_async_copy(tab_hbm.at[0].at[wsv.at[b]],
                                  rows.at[buf], sem.at[buf]).wait()

            def grp(g, _):
                d16 = wdv[pl.ds(b * 128 + g * 16, 16)]
                for j in range(16):
                    dl = d16[j]
                    e = g * 16 + j
                    for q in range(8):
                        sl = pl.ds(q * 16, 16)
                        asum[dl, sl] = asum[dl, sl] + rows[buf, e, sl]
                return 0
            lax.fori_loop(0, 8, grp, 0)
            return 0
        lax.fori_loop(0, nb, batch0, 0)
        pltpu.sync_copy(asum.at[pl.ds(0, BKT)], nsx_hbm.at[pl.ds(base, BKT)])

        def tower(ch, _):
            def zacc(i, _):
                for q in range(8):
                    asum[i, pl.ds(q * 16, 16)] = zero
                    amin[i, pl.ds(q * 16, 16)] = big
                    amax[i, pl.ds(q * 16, 16)] = -big
                return 0
            lax.fori_loop(0, 168, zacc, 0)

            @pl.when(nb > 0)
            def _():
                pltpu.async_copy(tab_hbm.at[ch].at[wsv.at[0]], rows.at[0],
                                 sem.at[0])

            def batch(b, _):
                buf = b % 2

                @pl.when(b + 1 < nb)
                def _():
                    nxt = (b + 1) % 2
                    pltpu.async_copy(tab_hbm.at[ch].at[wsv.at[b + 1]],
                                     rows.at[nxt], sem.at[nxt])
                pltpu.make_async_copy(tab_hbm.at[ch].at[wsv.at[b]],
                                      rows.at[buf], sem.at[buf]).wait()

                def grp(g, _):
                    d16 = wdv[pl.ds(b * 128 + g * 16, 16)]
                    for j in range(16):
                        dl = d16[j]
                        e = g * 16 + j
                        for q in range(8):
                            sl = pl.ds(q * 16, 16)
                            v = rows[buf, e, sl]
                            amin[dl, sl] = jnp.minimum(amin[dl, sl], v)
                            amax[dl, sl] = jnp.maximum(amax[dl, sl], v)
                            asum[dl, sl] = asum[dl, sl] + v * v
                    return 0
                lax.fori_loop(0, 8, grp, 0)
                return 0
            lax.fori_loop(0, nb, batch, 0)

            t = ch - 1
            pltpu.sync_copy(asum.at[pl.ds(0, BKT)],
                            ssq_hbm.at[t].at[pl.ds(base, BKT)])
            pltpu.sync_copy(amin.at[pl.ds(0, BKT)],
                            mn_hbm.at[t].at[pl.ds(base, BKT)])
            pltpu.sync_copy(amax.at[pl.ds(0, BKT)],
                            mx_hbm.at[t].at[pl.ds(base, BKT)])
            return 0
        lax.fori_loop(1, NCHUNK, tower, 0)
        return 0
    lax.fori_loop(0, 2, pass_body, 0)


def _pad128(a):
    n, f = a.shape
    if f == 128:
        return a
    return jnp.pad(a, ((0, 0), (0, 128 - f)))


def _pna_layer(h, p, wsrc, wdst, cnts, deg, avg_log):
    n, Fi = h.shape
    W1 = p['Wpre'][:, :Fi, :]
    W2 = p['Wpre'][:, Fi:, :]
    Bt = jnp.einsum('nf,tfg->tng', h, W2)
    tabs = jnp.concatenate([_pad128(h)[None],
                            jax.vmap(_pad128)(Bt)], axis=0)
    nsx, ssq, mnb, mxb = _segment_kernel(tabs, wsrc, wdst, cnts)
    NSx = nsx[:n, :Fi]
    degc = jnp.clip(deg, 1.0, None)
    logd = jnp.log(degc + 1.0)[:, None]
    r1 = logd / avg_log
    r2 = avg_log / logd
    dcol = deg[:, None]
    has = dcol > 0
    outs = []
    for t in range(T):
        c = h @ W1[t] + p['bpre'][t]
        sumB = NSx @ W2[t]
        s = dcol * c + sumB
        mean = s / degc[:, None]
        s2 = dcol * c * c + 2.0 * c * sumB + ssq[t, :n, :Fi]
        var = jnp.maximum(s2 / degc[:, None] - mean * mean, 0.0)
        std = jnp.sqrt(var + 1e-5)
        mn = jnp.where(has, c + mnb[t, :n, :Fi], 0.0)
        mx = jnp.where(has, c + mxb[t, :n, :Fi], 0.0)
        agg = jnp.concatenate([mean, mn, mx, std], axis=-1)
        out = jnp.concatenate([h, agg, agg * r1, agg * r2], axis=-1)
        outs.append(out @ p['Wpost'][t] + p['bpost'][t])
    o = jnp.concatenate(outs, axis=-1)
    o = o @ p['Wlin'] + p['blin']
    m = jnp.mean(o, axis=0)
    v = jnp.var(o, axis=0)
    o = p['gamma'] * (o - m) / jnp.sqrt(v + 1e-5) + p['beta']
    return jax.nn.relu(o)


def kernel(x, edge_index, batch, params):
    src = edge_index[0]
    dst = edge_index[1]
    wsrc, wdst, cnts, degr = _filter_kernel(src, dst)
    deg = degr[:, :320, 0].reshape(NP)[:N]

    h = x
    for l in range(NL):
        h = _pna_layer(h, params['conv%d' % l], wsrc, wdst, cnts, deg,
                       AVG_LOG_CONST)

    onehot = (batch[:, None] == jnp.arange(G)[None, :]).astype(h.dtype)
    s = onehot.T @ h
    c = jnp.sum(onehot, axis=0)
    g = s / jnp.clip(c, 1.0, None)[:, None]
    m = params['mlp']
    g = jax.nn.relu(g @ m['W1'] + m['b1'])
    g = jax.nn.relu(g @ m['W2'] + m['b2'])
    return g @ m['W3'] + m['b3']


# speedup vs baseline: 4.4890x; 1.2421x over previous
import functools
import math

import jax
import jax.numpy as jnp
import numpy as np
from jax import lax
from jax.experimental import pallas as pl
from jax.experimental.pallas import tpu as pltpu
from jax.experimental.pallas import tpu_sc as plsc

N = 10000
E = 320000
D = 128
H = 100
T = 5
G = 64
NL = 3
DEG_HIST = np.zeros(33, dtype=np.float64)
DEG_HIST[32] = N
_bins = np.arange(33, dtype=np.float64)
AVG_LOG_CONST = float((np.log(_bins + 1.0) * DEG_HIST).sum() / DEG_HIST.sum())

NW = 32
NBKT = 64
BKT = 160
NP = NBKT * BKT
CAP = 8192
NB = CAP // 128
CH = 8000
NCHUNK = 1 + T

_mesh = plsc.VectorSubcoreMesh(core_axis_name="c", subcore_axis_name="s")


PSEG = 448
PCAP = 96
SCCH = E // 16 // 2


@functools.partial(
    pl.kernel,
    out_type=(
        jax.ShapeDtypeStruct((NBKT, NB, 128), jnp.int32),
        jax.ShapeDtypeStruct((NBKT, 16), jnp.int32),
        jax.ShapeDtypeStruct((NW, 5120), jnp.float32),
    ),
    mesh=_mesh,
    scratch_types=[
        pltpu.VMEM((SCCH,), jnp.int32),
        pltpu.VMEM((SCCH,), jnp.int32),
        pltpu.VMEM((16 * PSEG,), jnp.int32),
        pltpu.VMEM((320 * PCAP,), jnp.int32),
        pltpu.VMEM((16, 16), jnp.int32),
        pltpu.VMEM((PSEG,), jnp.int32),
        pltpu.VMEM((328, 16), jnp.int32),
        pltpu.VMEM((5120,), jnp.float32),
        pltpu.VMEM((CAP + 128,), jnp.int32),
        pltpu.VMEM((16,), jnp.int32),
        pltpu.VMEM_SHARED((16 * 16 * PSEG,), jnp.int32),
        pltpu.VMEM_SHARED((16 * 16 * 16,), jnp.int32),
        pltpu.SemaphoreType.DMA,
    ],
)
def _filter_kernel(src_hbm, dst_hbm, wsrc_hbm, cnt_hbm, deg_hbm,
                   sv, dv, pbin, pbin2, c16, segs, cntb, degf,
                   wso, cstg, sh_s, sh_c, sem):
    c = lax.axis_index("c")
    s = lax.axis_index("s")
    zi = jnp.zeros((16,), jnp.int32)

    lo = (c * 16 + s) * 320

    def zb(i, _):
        cntb[i, :] = zi
        return 0
    lax.fori_loop(0, 328, zb, 0)

    def rnd(r, _):
        def zc(i, _):
            c16[i, :] = zi
            return 0
        lax.fori_loop(0, 16, zc, 0)

        ebase = pl.multiple_of(s * (E // 16) + r * SCCH, 8)
        pltpu.sync_copy(src_hbm.at[pl.ds(ebase, SCCH)], sv)
        pltpu.sync_copy(dst_hbm.at[pl.ds(ebase, SCCH)], dv)

        def vec(i, _):
            d16 = dv[pl.ds(i * 16, 16)]
            s16 = sv[pl.ds(i * 16, 16)]
            for j in range(16):
                dj = d16[j]
                ol = dj // 320 - c * 16
                keep = (ol >= 0) & (ol < 16)
                olc = jnp.clip(ol, 0, 15)
                cvec = c16[olc, :]
                cc = cvec[0]
                ok = keep & (cc < PSEG - 16)
                addr = olc * PSEG + cc

                @pl.when(ok)
                def _():
                    pbin[pl.ds(addr, 16)] = \
                        jnp.broadcast_to(s16[j] | (dj << 14), (16,))
                    c16[olc, :] = cvec + 1
            return 0
        lax.fori_loop(0, SCCH // 16, vec, 0)

        for o in range(16):
            pltpu.sync_copy(pbin.at[pl.ds(o * PSEG, PSEG)],
                            sh_s.at[pl.ds((s * 16 + o) * PSEG, PSEG)])
        for o in range(16):
            pltpu.sync_copy(c16.at[o], sh_c.at[pl.ds((s * 16 + o) * 16, 16)])
        plsc.subcore_barrier()

        def seg(scn, _):
            sbase = pl.multiple_of((scn * 16 + s) * 16, 8)
            pltpu.sync_copy(sh_c.at[pl.ds(sbase, 16)], cstg)
            m = cstg[pl.ds(0, 16)][0]
            pbase = pl.multiple_of((scn * 16 + s) * PSEG, 8)
            pltpu.sync_copy(sh_s.at[pl.ds(pbase, PSEG)], segs)

            def grp(g, _):
                s16 = segs[pl.ds(g * 16, 16)]
                for j in range(16):
                    v = s16[j]
                    dl = lax.shift_right_logical(v, 14) - lo
                    dlc = jnp.clip(dl, 0, 319)
                    cvec = cntb[dlc, :]
                    cc = cvec[0]
                    ok = (g * 16 + j < m) & (cc < PCAP - 16)
                    addr = dlc * PCAP + cc

                    @pl.when(ok)
                    def _():
                        pbin2[pl.ds(addr, 16)] = \
                            jnp.broadcast_to(v & 16383, (16,))
                        cntb[dlc, :] = cvec + 1
                return 0
            return lax.fori_loop(0, (m + 15) // 16, grp, 0)
        lax.fori_loop(0, 16, seg, 0)
        plsc.subcore_barrier()
        return 0
    lax.fori_loop(0, 2, rnd, 0)

    def cdeg(i, _):
        degf[pl.ds(i * 16, 16)] = cntb[i, :].astype(jnp.float32)
        return 0
    lax.fori_loop(0, 320, cdeg, 0)
    pltpu.sync_copy(degf, deg_hbm.at[c * 16 + s])

    for p in range(2):
        bkt = 2 * (c * 16 + s) + p

        def bin_copy(r, pos):
            cvec = cntb[p * BKT + r, :]
            m = cvec[0]
            fits = pos + PCAP <= CAP
            me = jnp.where(fits, m, 0)

            @pl.when(me > 0)
            def _():
                rtag = jnp.full((16,), r << 16, jnp.int32)

                def cp(k, _):
                    wso[pl.ds(pos + k * 16, 16)] = rtag | \
                        pbin2[pl.ds((p * BKT + r) * PCAP + k * 16, 16)]
                    return 0
                lax.fori_loop(0, (me + 15) // 16, cp, 0)
            return pos + me
        cnt = lax.fori_loop(0, BKT, bin_copy, 0)

        for t in range(8):
            wso[pl.ds(cnt + t * 16, 16)] = jnp.full((16,), BKT << 16,
                                                    jnp.int32)

        for b in range(NB):
            pltpu.sync_copy(wso.at[pl.ds(b * 128, 128)],
                            wsrc_hbm.at[bkt].at[b])
        cstg[pl.ds(0, 16)] = jnp.broadcast_to(cnt, (16,))
        pltpu.sync_copy(cstg, cnt_hbm.at[bkt])


@functools.partial(
    pl.kernel,
    out_type=(
        jax.ShapeDtypeStruct((NP, 128), jnp.float32),
        jax.ShapeDtypeStruct((T, NP, 128), jnp.float32),
        jax.ShapeDtypeStruct((T, NP, 128), jnp.float32),
        jax.ShapeDtypeStruct((T, NP, 128), jnp.float32),
    ),
    mesh=_mesh,
    scratch_types=[
        pltpu.VMEM((NB, 128), jnp.int32),
        pltpu.VMEM((CAP,), jnp.int32),
        pltpu.VMEM((16,), jnp.int32),
        pltpu.VMEM((2, 128, 128), jnp.float32),
        pltpu.VMEM((168, 128), jnp.float32),
        pltpu.VMEM((168, 128), jnp.float32),
        pltpu.VMEM((168, 128), jnp.float32),
        pltpu.SemaphoreType.DMA((2,)),
    ],
)
def _segment_kernel(tab_hbm, wsrc_hbm, cnt_hbm,
                    nsx_hbm, ssq_hbm, mn_hbm, mx_hbm,
                    wsv, wdv, cntv, rows, asum, amin, amax, sem):
    w = lax.axis_index("s") * 2 + lax.axis_index("c")
    zero = jnp.zeros((16,), jnp.float32)
    big = jnp.full((16,), 3e38, jnp.float32)

    def pass_body(p, _):
        bkt = 2 * w + p
        base = pl.multiple_of(bkt * BKT, 8)
        pltpu.sync_copy(wsrc_hbm.at[bkt], wsv)
        pltpu.sync_copy(cnt_hbm.at[bkt], cntv)
        cnt = cntv[pl.ds(0, 16)][0]
        nb = (cnt + 127) // 128

        def unpack(b, _):
            for k in range(8):
                sl = pl.ds(k * 16, 16)
                v = wsv[b, sl]
                wdv[pl.ds(b * 128 + k * 16, 16)] = \
                    lax.shift_right_logical(v, 16)
                wsv[b, sl] = v & 0xFFFF
            return 0
        lax.fori_loop(0, nb, unpack, 0)

        def zsum(i, _):
            for q in range(8):
                asum[i, pl.ds(q * 16, 16)] = zero
            return 0
        lax.fori_loop(0, 168, zsum, 0)

        @pl.when(nb > 0)
        def _():
            pltpu.async_copy(tab_hbm.at[0].at[wsv.at[0]], rows.at[0],
                             sem.at[0])

        def batch0(b, _):
            buf = b % 2

            @pl.when(b + 1 < nb)
            def _():
                nxt = (b + 1) % 2
                pltpu.async_copy(tab_hbm.at[0].at[wsv.at[b + 1]],
                                 rows.at[nxt], sem.at[nxt])
            pltpu.make_async_copy(tab_hbm.at[0].at[wsv.at[b]],
                                  rows.at[buf], sem.at[buf]).wait()

            def grp(g, _):
                d16 = wdv[pl.ds(b * 128 + g * 16, 16)]
                for j in range(16):
                    dl = d16[j]
                    e = g * 16 + j
                    for q in range(8):
                        sl = pl.ds(q * 16, 16)
                        asum[dl, sl] = asum[dl, sl] + rows[buf, e, sl]
                return 0
            lax.fori_loop(0, 8, grp, 0)
            return 0
        lax.fori_loop(0, nb, batch0, 0)
        pltpu.sync_copy(asum.at[pl.ds(0, BKT)], nsx_hbm.at[pl.ds(base, BKT)])

        def tower(ch, _):
            def zacc(i, _):
                for q in range(8):
                    asum[i, pl.ds(q * 16, 16)] = zero
                    amin[i, pl.ds(q * 16, 16)] = big
                    amax[i, pl.ds(q * 16, 16)] = -big
                return 0
            lax.fori_loop(0, 168, zacc, 0)

            @pl.when(nb > 0)
            def _():
                pltpu.async_copy(tab_hbm.at[ch].at[wsv.at[0]], rows.at[0],
                                 sem.at[0])

            def batch(b, _):
                buf = b % 2

                @pl.when(b + 1 < nb)
                def _():
                    nxt = (b + 1) % 2
                    pltpu.async_copy(tab_hbm.at[ch].at[wsv.at[b + 1]],
                                     rows.at[nxt], sem.at[nxt])
                pltpu.make_async_copy(tab_hbm.at[ch].at[wsv.at[b]],
                                      rows.at[buf], sem.at[buf]).wait()

                def grp(g, _):
                    d16 = wdv[pl.ds(b * 128 + g * 16, 16)]
                    for j in range(16):
                        dl = d16[j]
                        e = g * 16 + j
                        for q in range(8):
                            sl = pl.ds(q * 16, 16)
                            v = rows[buf, e, sl]
                            amin[dl, sl] = jnp.minimum(amin[dl, sl], v)
                            amax[dl, sl] = jnp.maximum(amax[dl, sl], v)
                            asum[dl, sl] = asum[dl, sl] + v * v
                    return 0
                lax.fori_loop(0, 8, grp, 0)
                return 0
            lax.fori_loop(0, nb, batch, 0)

            t = ch - 1
            pltpu.sync_copy(asum.at[pl.ds(0, BKT)],
                            ssq_hbm.at[t].at[pl.ds(base, BKT)])
            pltpu.sync_copy(amin.at[pl.ds(0, BKT)],
                            mn_hbm.at[t].at[pl.ds(base, BKT)])
            pltpu.sync_copy(amax.at[pl.ds(0, BKT)],
                            mx_hbm.at[t].at[pl.ds(base, BKT)])
            return 0
        lax.fori_loop(1, NCHUNK, tower, 0)
        return 0
    lax.fori_loop(0, 2, pass_body, 0)


def _pad128(a):
    n, f = a.shape
    if f == 128:
        return a
    return jnp.pad(a, ((0, 0), (0, 128 - f)))


def _pna_layer(h, p, wsrc, cnts, deg, avg_log):
    n, Fi = h.shape
    W1 = p['Wpre'][:, :Fi, :]
    W2 = p['Wpre'][:, Fi:, :]
    Bt = jnp.einsum('nf,tfg->tng', h, W2)
    tabs = jnp.concatenate([_pad128(h)[None],
                            jax.vmap(_pad128)(Bt)], axis=0)
    nsx, ssq, mnb, mxb = _segment_kernel(tabs, wsrc, cnts)
    NSx = nsx[:n, :Fi]
    degc = jnp.clip(deg, 1.0, None)
    logd = jnp.log(degc + 1.0)[:, None]
    r1 = logd / avg_log
    r2 = avg_log / logd
    dcol = deg[:, None]
    has = dcol > 0
    outs = []
    for t in range(T):
        c = h @ W1[t] + p['bpre'][t]
        sumB = NSx @ W2[t]
        s = dcol * c + sumB
        mean = s / degc[:, None]
        s2 = dcol * c * c + 2.0 * c * sumB + ssq[t, :n, :Fi]
        var = jnp.maximum(s2 / degc[:, None] - mean * mean, 0.0)
        std = jnp.sqrt(var + 1e-5)
        mn = jnp.where(has, c + mnb[t, :n, :Fi], 0.0)
        mx = jnp.where(has, c + mxb[t, :n, :Fi], 0.0)
        agg = jnp.concatenate([mean, mn, mx, std], axis=-1)
        out = jnp.concatenate([h, agg, agg * r1, agg * r2], axis=-1)
        outs.append(out @ p['Wpost'][t] + p['bpost'][t])
    o = jnp.concatenate(outs, axis=-1)
    o = o @ p['Wlin'] + p['blin']
    m = jnp.mean(o, axis=0)
    v = jnp.var(o, axis=0)
    o = p['gamma'] * (o - m) / jnp.sqrt(v + 1e-5) + p['beta']
    return jax.nn.relu(o)


def kernel(x, edge_index, batch, params):
    src = edge_index[0]
    dst = edge_index[1]
    wsrc, cnts, degr = _filter_kernel(src, dst)
    deg = degr.reshape(NW, 320, 16)[:, :, 0].reshape(NP)[:N]

    h = x
    for l in range(NL):
        h = _pna_layer(h, params['conv%d' % l], wsrc, cnts, deg,
                       AVG_LOG_CONST)

    onehot = (batch[:, None] == jnp.arange(G)[None, :]).astype(h.dtype)
    s = onehot.T @ h
    c = jnp.sum(onehot, axis=0)
    g = s / jnp.clip(c, 1.0, None)[:, None]
    m = params['mlp']
    g = jax.nn.relu(g @ m['W1'] + m['b1'])
    g = jax.nn.relu(g @ m['W2'] + m['b2'])
    return g @ m['W3'] + m['b3']


# speedup vs baseline: 11.0874x; 2.4699x over previous
import functools
import math

import jax
import jax.numpy as jnp
import numpy as np
from jax import lax
from jax.experimental import pallas as pl
from jax.experimental.pallas import tpu as pltpu
from jax.experimental.pallas import tpu_sc as plsc

N = 10000
E = 320000
D = 128
H = 100
T = 5
G = 64
NL = 3
DEG_HIST = np.zeros(33, dtype=np.float64)
DEG_HIST[32] = N
_bins = np.arange(33, dtype=np.float64)
AVG_LOG_CONST = float((np.log(_bins + 1.0) * DEG_HIST).sum() / DEG_HIST.sum())

NW = 32
NBKT = 64
BKT = 160
NP = NBKT * BKT
CAP = 8192
NB = CAP // 128
CH = 8000
NCHUNK = 1 + T

_mesh = plsc.VectorSubcoreMesh(core_axis_name="c", subcore_axis_name="s")


PSEG = 448
PCAP = 96
SCCH = E // 16 // 2


@functools.partial(
    pl.kernel,
    out_type=(
        jax.ShapeDtypeStruct((NBKT, NB, 128), jnp.int32),
        jax.ShapeDtypeStruct((NBKT, 256), jnp.int32),
        jax.ShapeDtypeStruct((NBKT, 16), jnp.int32),
        jax.ShapeDtypeStruct((NW, 5120), jnp.float32),
    ),
    mesh=_mesh,
    scratch_types=[
        pltpu.VMEM((SCCH,), jnp.int32),
        pltpu.VMEM((SCCH,), jnp.int32),
        pltpu.VMEM((16 * PSEG,), jnp.int32),
        pltpu.VMEM((320 * PCAP,), jnp.int32),
        pltpu.VMEM((16, 16), jnp.int32),
        pltpu.VMEM((PSEG,), jnp.int32),
        pltpu.VMEM((328, 16), jnp.int32),
        pltpu.VMEM((5120,), jnp.float32),
        pltpu.VMEM((CAP + 128,), jnp.int32),
        pltpu.VMEM((256,), jnp.int32),
        pltpu.VMEM((16,), jnp.int32),
        pltpu.VMEM_SHARED((16 * 16 * PSEG,), jnp.int32),
        pltpu.VMEM_SHARED((16 * 16 * 16,), jnp.int32),
        pltpu.SemaphoreType.DMA,
    ],
)
def _filter_kernel(src_hbm, dst_hbm, wsrc_hbm, off_hbm, cnt_hbm, deg_hbm,
                   sv, dv, pbin, pbin2, c16, segs, cntb, degf,
                   wso, offv, cstg, sh_s, sh_c, sem):
    c = lax.axis_index("c")
    s = lax.axis_index("s")
    zi = jnp.zeros((16,), jnp.int32)

    lo = (c * 16 + s) * 320

    def zb(i, _):
        cntb[i, :] = zi
        return 0
    lax.fori_loop(0, 328, zb, 0)

    def rnd(r, _):
        def zc(i, _):
            c16[i, :] = zi
            return 0
        lax.fori_loop(0, 16, zc, 0)

        ebase = pl.multiple_of(s * (E // 16) + r * SCCH, 8)
        pltpu.sync_copy(src_hbm.at[pl.ds(ebase, SCCH)], sv)
        pltpu.sync_copy(dst_hbm.at[pl.ds(ebase, SCCH)], dv)

        def vec(i, _):
            d16 = dv[pl.ds(i * 16, 16)]
            s16 = sv[pl.ds(i * 16, 16)]
            for j in range(16):
                dj = d16[j]
                ol = dj // 320 - c * 16
                keep = (ol >= 0) & (ol < 16)
                olc = jnp.clip(ol, 0, 15)
                cvec = c16[olc, :]
                cc = cvec[0]
                ok = keep & (cc < PSEG - 16)
                addr = olc * PSEG + cc

                @pl.when(ok)
                def _():
                    pbin[pl.ds(addr, 16)] = \
                        jnp.broadcast_to(s16[j] | (dj << 14), (16,))
                    c16[olc, :] = cvec + 1
            return 0
        lax.fori_loop(0, SCCH // 16, vec, 0)

        for o in range(16):
            pltpu.sync_copy(pbin.at[pl.ds(o * PSEG, PSEG)],
                            sh_s.at[pl.ds((s * 16 + o) * PSEG, PSEG)])
        for o in range(16):
            pltpu.sync_copy(c16.at[o], sh_c.at[pl.ds((s * 16 + o) * 16, 16)])
        plsc.subcore_barrier()

        def seg(scn, _):
            sbase = pl.multiple_of((scn * 16 + s) * 16, 8)
            pltpu.sync_copy(sh_c.at[pl.ds(sbase, 16)], cstg)
            m = cstg[pl.ds(0, 16)][0]
            pbase = pl.multiple_of((scn * 16 + s) * PSEG, 8)
            pltpu.sync_copy(sh_s.at[pl.ds(pbase, PSEG)], segs)

            def grp(g, _):
                s16 = segs[pl.ds(g * 16, 16)]
                for j in range(16):
                    v = s16[j]
                    dl = lax.shift_right_logical(v, 14) - lo
                    dlc = jnp.clip(dl, 0, 319)
                    cvec = cntb[dlc, :]
                    cc = cvec[0]
                    ok = (g * 16 + j < m) & (cc < PCAP - 16)
                    addr = dlc * PCAP + cc

                    @pl.when(ok)
                    def _():
                        pbin2[pl.ds(addr, 16)] = \
                            jnp.broadcast_to(v & 16383, (16,))
                        cntb[dlc, :] = cvec + 1
                return 0
            return lax.fori_loop(0, (m + 15) // 16, grp, 0)
        lax.fori_loop(0, 16, seg, 0)
        plsc.subcore_barrier()
        return 0
    lax.fori_loop(0, 2, rnd, 0)

    def cdeg(i, _):
        degf[pl.ds(i * 16, 16)] = cntb[i, :].astype(jnp.float32)
        return 0
    lax.fori_loop(0, 320, cdeg, 0)
    pltpu.sync_copy(degf, deg_hbm.at[c * 16 + s])

    for p in range(2):
        bkt = 2 * (c * 16 + s) + p

        def bin_copy(r, pos):
            offv[pl.ds(r, 16)] = jnp.broadcast_to(pos, (16,))
            cvec = cntb[p * BKT + r, :]
            m = cvec[0]
            fits = pos + PCAP <= CAP
            me = jnp.where(fits, m, 0)

            @pl.when(me > 0)
            def _():
                rtag = jnp.full((16,), r << 16, jnp.int32)

                def cp(k, _):
                    wso[pl.ds(pos + k * 16, 16)] = rtag | \
                        pbin2[pl.ds((p * BKT + r) * PCAP + k * 16, 16)]
                    return 0
                lax.fori_loop(0, (me + 15) // 16, cp, 0)
            return pos + me
        cnt = lax.fori_loop(0, BKT, bin_copy, 0)

        offv[pl.ds(BKT, 16)] = jnp.broadcast_to(cnt, (16,))
        for t in range(8):
            wso[pl.ds(cnt + t * 16, 16)] = jnp.full((16,), BKT << 16,
                                                    jnp.int32)

        pltpu.sync_copy(offv, off_hbm.at[bkt])
        for b in range(NB):
            pltpu.sync_copy(wso.at[pl.ds(b * 128, 128)],
                            wsrc_hbm.at[bkt].at[b])
        cstg[pl.ds(0, 16)] = jnp.broadcast_to(cnt, (16,))
        pltpu.sync_copy(cstg, cnt_hbm.at[bkt])


@functools.partial(
    pl.kernel,
    out_type=(
        jax.ShapeDtypeStruct((NP, 128), jnp.float32),
        jax.ShapeDtypeStruct((T, NP, 128), jnp.float32),
        jax.ShapeDtypeStruct((T, NP, 128), jnp.float32),
        jax.ShapeDtypeStruct((T, NP, 128), jnp.float32),
    ),
    mesh=_mesh,
    scratch_types=[
        pltpu.VMEM((NB, 128), jnp.int32),
        pltpu.VMEM((256,), jnp.int32),
        pltpu.VMEM((16,), jnp.int32),
        pltpu.VMEM((2, 128, 128), jnp.float32),
        pltpu.VMEM((168, 128), jnp.float32),
        pltpu.VMEM((168, 128), jnp.float32),
        pltpu.VMEM((168, 128), jnp.float32),
        pltpu.SemaphoreType.DMA((2,)),
    ],
)
def _segment_kernel(tab_hbm, wsrc_hbm, off_hbm, cnt_hbm,
                    nsx_hbm, ssq_hbm, mn_hbm, mx_hbm,
                    wsv, offv, cntv, rows, asum, amin, amax, sem):
    w = lax.axis_index("s") * 2 + lax.axis_index("c")
    zero = jnp.zeros((16,), jnp.float32)
    big = jnp.full((16,), 3e38, jnp.float32)

    def pass_body(p, _):
        bkt = 2 * w + p
        base = pl.multiple_of(bkt * BKT, 8)
        pltpu.sync_copy(wsrc_hbm.at[bkt], wsv)
        pltpu.sync_copy(off_hbm.at[bkt], offv)
        pltpu.sync_copy(cnt_hbm.at[bkt], cntv)
        cnt = cntv[pl.ds(0, 16)][0]
        nb = (cnt + 127) // 128

        def unpack(b, _):
            for k in range(8):
                sl = pl.ds(k * 16, 16)
                wsv[b, sl] = wsv[b, sl] & 0xFFFF
            return 0
        lax.fori_loop(0, nb, unpack, 0)

        def chunk_loop(ch, _):
            @pl.when(nb > 0)
            def _():
                pltpu.async_copy(tab_hbm.at[ch].at[wsv.at[0]], rows.at[0],
                                 sem.at[0])

            def node_sum(nd, cur_b):
                o16 = offv[pl.ds(nd, 16)]
                start = o16[0]
                end = o16[1]

                def edge(e, carry):
                    cur_b, r0, r1, r2, r3, r4, r5, r6, r7 = carry
                    eb = e >> 7

                    @pl.when(eb != cur_b)
                    def _():
                        @pl.when(eb + 1 < nb)
                        def _():
                            nxt = (eb + 1) % 2
                            pltpu.async_copy(
                                tab_hbm.at[ch].at[wsv.at[eb + 1]],
                                rows.at[nxt], sem.at[nxt])
                        pltpu.make_async_copy(
                            tab_hbm.at[ch].at[wsv.at[eb]],
                            rows.at[eb % 2], sem.at[eb % 2]).wait()
                    buf = eb % 2
                    sl0 = e & 127
                    rs = [r0, r1, r2, r3, r4, r5, r6, r7]
                    out = []
                    for q in range(8):
                        v = rows[buf, sl0, pl.ds(q * 16, 16)]
                        out.append(rs[q] + v)
                    return (eb,) + tuple(out)
                carry = (cur_b,) + (zero,) * 8
                carry = lax.fori_loop(start, end, edge, carry)
                cur_b = carry[0]
                for q in range(8):
                    asum[nd, pl.ds(q * 16, 16)] = carry[1 + q]
                return cur_b

            def node_tower(nd, cur_b):
                o16 = offv[pl.ds(nd, 16)]
                start = o16[0]
                end = o16[1]

                def edge(e, carry):
                    cur_b = carry[0]
                    eb = e >> 7

                    @pl.when(eb != cur_b)
                    def _():
                        @pl.when(eb + 1 < nb)
                        def _():
                            nxt = (eb + 1) % 2
                            pltpu.async_copy(
                                tab_hbm.at[ch].at[wsv.at[eb + 1]],
                                rows.at[nxt], sem.at[nxt])
                        pltpu.make_async_copy(
                            tab_hbm.at[ch].at[wsv.at[eb]],
                            rows.at[eb % 2], sem.at[eb % 2]).wait()
                    buf = eb % 2
                    sl0 = e & 127
                    out = []
                    for q in range(8):
                        v = rows[buf, sl0, pl.ds(q * 16, 16)]
                        rm, rx, rq = carry[1 + 3 * q: 4 + 3 * q]
                        out += [jnp.minimum(rm, v), jnp.maximum(rx, v),
                                rq + v * v]
                    return (eb,) + tuple(out)
                carry = (cur_b,) + (big, -big, zero) * 8
                carry = lax.fori_loop(start, end, edge, carry)
                cur_b = carry[0]
                for q in range(8):
                    sl = pl.ds(q * 16, 16)
                    amin[nd, sl] = carry[1 + 3 * q]
                    amax[nd, sl] = carry[2 + 3 * q]
                    asum[nd, sl] = carry[3 + 3 * q]
                return cur_b

            @pl.when(ch == 0)
            def _():
                lax.fori_loop(0, BKT, node_sum, -1)
                pltpu.sync_copy(asum.at[pl.ds(0, BKT)],
                                nsx_hbm.at[pl.ds(base, BKT)])

            @pl.when(ch > 0)
            def _():
                lax.fori_loop(0, BKT, node_tower, -1)
                t = ch - 1
                pltpu.sync_copy(asum.at[pl.ds(0, BKT)],
                                ssq_hbm.at[t].at[pl.ds(base, BKT)])
                pltpu.sync_copy(amin.at[pl.ds(0, BKT)],
                                mn_hbm.at[t].at[pl.ds(base, BKT)])
                pltpu.sync_copy(amax.at[pl.ds(0, BKT)],
                                mx_hbm.at[t].at[pl.ds(base, BKT)])
            return 0
        lax.fori_loop(0, NCHUNK, chunk_loop, 0)
        return 0
    lax.fori_loop(0, 2, pass_body, 0)


RBLK = 400
NBLK = N // RBLK


@functools.cache
def _tables_kernel(Fi, entry_norm):
    def body(h_ref, st_ref, w2_ref, tab_ref):
        hv = h_ref[...]
        if entry_norm:
            m = st_ref[0, :Fi][None, :]
            v = st_ref[1, :Fi][None, :]
            gm = st_ref[2, :Fi][None, :]
            bt = st_ref[3, :Fi][None, :]
            hv = jnp.maximum(gm * (hv - m) / jnp.sqrt(v + 1e-5) + bt, 0.0)
        def padded(a):
            if Fi == 128:
                return a
            return jnp.concatenate(
                [a, jnp.zeros((RBLK, 128 - Fi), jnp.float32)], axis=1)
        tab_ref[0] = padded(hv)
        for t in range(T):
            tab_ref[1 + t] = padded(hv @ w2_ref[t])

    return pl.pallas_call(
        body,
        grid=(NBLK,),
        in_specs=[
            pl.BlockSpec((RBLK, Fi), lambda i: (i, 0)),
            pl.BlockSpec((4, 128), lambda i: (0, 0)),
            pl.BlockSpec((T, Fi, Fi), lambda i: (0, 0, 0)),
        ],
        out_specs=pl.BlockSpec((1 + T, RBLK, 128), lambda i: (0, i, 0)),
        out_shape=jax.ShapeDtypeStruct((1 + T, N, 128), jnp.float32),
    )


@functools.cache
def _dense_kernel(Fi, entry_norm):
    def body(h_ref, st_ref, scal_ref, nsx_ref, ssq_ref, mn_ref, mx_ref,
             wpre_ref, bpre_ref, wpost_ref, bpost_ref, wlin_ref, blin_ref,
             o_ref, ps_ref, pq_ref):
        hv = h_ref[...]
        if entry_norm:
            m = st_ref[0, :Fi][None, :]
            v = st_ref[1, :Fi][None, :]
            gm = st_ref[2, :Fi][None, :]
            bt = st_ref[3, :Fi][None, :]
            hv = jnp.maximum(gm * (hv - m) / jnp.sqrt(v + 1e-5) + bt, 0.0)
        scal = scal_ref[...]
        deg = scal[:, 0:1]
        degc = scal[:, 1:2]
        r1 = scal[:, 2:3]
        r2 = scal[:, 3:4]
        has = deg > 0.0
        nsx = nsx_ref[...][:, :Fi]
        outs = []
        for t in range(T):
            W1 = wpre_ref[t, :Fi, :]
            W2 = wpre_ref[t, Fi:, :]
            c = hv @ W1 + bpre_ref[0, t, :][None, :]
            sumB = nsx @ W2
            sq = ssq_ref[t][:, :Fi]
            sv = deg * c + sumB
            mean = sv / degc
            s2 = deg * c * c + 2.0 * c * sumB + sq
            var = jnp.maximum(s2 / degc - mean * mean, 0.0)
            std = jnp.sqrt(var + 1e-5)
            mnv = jnp.where(has, c + mn_ref[t][:, :Fi], 0.0)
            mxv = jnp.where(has, c + mx_ref[t][:, :Fi], 0.0)
            agg = jnp.concatenate([mean, mnv, mxv, std], axis=1)
            out = jnp.concatenate([hv, agg, agg * r1, agg * r2], axis=1)
            outs.append(out @ wpost_ref[t] + bpost_ref[0, t, :][None, :])
        o = jnp.concatenate(outs, axis=1) @ wlin_ref[...] + \
            blin_ref[0, :][None, :]
        o_ref[...] = o
        ps_ref[...] = jnp.sum(o, axis=0)[None, None, :]
        pq_ref[...] = jnp.sum(o * o, axis=0)[None, None, :]

    return pl.pallas_call(
        body,
        grid=(NBLK,),
        in_specs=[
            pl.BlockSpec((RBLK, Fi), lambda i: (i, 0)),
            pl.BlockSpec((4, 128), lambda i: (0, 0)),
            pl.BlockSpec((RBLK, 4), lambda i: (i, 0)),
            pl.BlockSpec((RBLK, 128), lambda i: (i, 0)),
            pl.BlockSpec((T, RBLK, 128), lambda i: (0, i, 0)),
            pl.BlockSpec((T, RBLK, 128), lambda i: (0, i, 0)),
            pl.BlockSpec((T, RBLK, 128), lambda i: (0, i, 0)),
            pl.BlockSpec((T, 2 * Fi, Fi), lambda i: (0, 0, 0)),
            pl.BlockSpec((1, T, Fi), lambda i: (0, 0, 0)),
            pl.BlockSpec((T, 13 * Fi, H // T), lambda i: (0, 0, 0)),
            pl.BlockSpec((1, T, H // T), lambda i: (0, 0, 0)),
            pl.BlockSpec((H, H), lambda i: (0, 0)),
            pl.BlockSpec((1, H), lambda i: (0, 0)),
        ],
        out_specs=[
            pl.BlockSpec((RBLK, H), lambda i: (i, 0)),
            pl.BlockSpec((1, 1, H), lambda i: (i, 0, 0)),
            pl.BlockSpec((1, 1, H), lambda i: (i, 0, 0)),
        ],
        out_shape=[
            jax.ShapeDtypeStruct((N, H), jnp.float32),
            jax.ShapeDtypeStruct((NBLK, 1, H), jnp.float32),
            jax.ShapeDtypeStruct((NBLK, 1, H), jnp.float32),
        ],
    )


def _readout_kernel():
    def body(o_ref, st_ref, oh_ref, w1_ref, b1_ref, w2_ref, b2_ref,
             w3_ref, b3_ref, out_ref, s_ref, c_ref):
        i = pl.program_id(0)

        @pl.when(i == 0)
        def _():
            s_ref[...] = jnp.zeros((G, H), jnp.float32)
            c_ref[...] = jnp.zeros((1, G), jnp.float32)
        m = st_ref[0, :H][None, :]
        v = st_ref[1, :H][None, :]
        gm = st_ref[2, :H][None, :]
        bt = st_ref[3, :H][None, :]
        hn = jnp.maximum(gm * (o_ref[...] - m) / jnp.sqrt(v + 1e-5) + bt,
                         0.0)
        oh = oh_ref[...]
        s_ref[...] += oh.T @ hn
        c_ref[...] += jnp.sum(oh, axis=0, keepdims=True)

        @pl.when(i == NBLK - 1)
        def _():
            cnt = jnp.maximum(c_ref[...], 1.0)
            g = s_ref[...] / cnt.T
            g = jnp.maximum(g @ w1_ref[...] + b1_ref[0, :][None, :], 0.0)
            g = jnp.maximum(g @ w2_ref[...] + b2_ref[0, :][None, :], 0.0)
            out_ref[...] = g @ w3_ref[...] + b3_ref[0, :][None, :]

    return pl.pallas_call(
        body,
        grid=(NBLK,),
        in_specs=[
            pl.BlockSpec((RBLK, H), lambda i: (i, 0)),
            pl.BlockSpec((4, 128), lambda i: (0, 0)),
            pl.BlockSpec((RBLK, G), lambda i: (i, 0)),
            pl.BlockSpec((H, 50), lambda i: (0, 0)),
            pl.BlockSpec((1, 50), lambda i: (0, 0)),
            pl.BlockSpec((50, 25), lambda i: (0, 0)),
            pl.BlockSpec((1, 25), lambda i: (0, 0)),
            pl.BlockSpec((25, 1), lambda i: (0, 0)),
            pl.BlockSpec((1, 1), lambda i: (0, 0)),
        ],
        out_specs=[
            pl.BlockSpec((G, 1), lambda i: (0, 0)),
            pl.BlockSpec((G, H), lambda i: (0, 0)),
            pl.BlockSpec((1, G), lambda i: (0, 0)),
        ],
        out_shape=[
            jax.ShapeDtypeStruct((G, 1), jnp.float32),
            jax.ShapeDtypeStruct((G, H), jnp.float32),
            jax.ShapeDtypeStruct((1, G), jnp.float32),
        ],
    )


def _stats(o, psum, psq, gamma, beta):
    m = jnp.sum(psum, axis=(0, 1)) / N
    v = jnp.sum(psq, axis=(0, 1)) / N - m * m
    st = jnp.zeros((4, 128), jnp.float32)
    st = st.at[0, :H].set(m).at[1, :H].set(v)
    st = st.at[2, :H].set(gamma).at[3, :H].set(beta)
    return st


def kernel(x, edge_index, batch, params):
    src = edge_index[0]
    dst = edge_index[1]
    wsrc, offs, cnts, degr = _filter_kernel(src, dst)
    deg = degr.reshape(NW, 320, 16)[:, :, 0].reshape(NP)[:N]

    degc = jnp.clip(deg, 1.0, None)
    logd = jnp.log(degc + 1.0)
    scal = jnp.stack([deg, degc, logd / AVG_LOG_CONST,
                      AVG_LOG_CONST / logd], axis=1)
    onehot = (batch[:, None] == jnp.arange(G)[None, :]).astype(jnp.float32)
    st_id = jnp.zeros((4, 128), jnp.float32).at[1, :].set(1.0)

    h = x
    st = st_id
    for l in range(NL):
        p = params['conv%d' % l]
        Fi = D if l == 0 else H
        norm = l > 0
        W2 = p['Wpre'][:, Fi:, :]
        tabs = _tables_kernel(Fi, norm)(h, st, W2)
        nsx, ssq, mnb, mxb = _segment_kernel(tabs, wsrc, offs, cnts)
        o, ps, pq = _dense_kernel(Fi, norm)(
            h, st, scal, nsx[:N], ssq[:, :N], mnb[:, :N], mxb[:, :N],
            p['Wpre'], p['bpre'][None], p['Wpost'], p['bpost'][None],
            p['Wlin'], p['blin'][None])
        st = _stats(o, ps, pq, p['gamma'], p['beta'])
        h = o

    m = params['mlp']
    out, _, _ = _readout_kernel()(
        h, st, onehot, m['W1'], m['b1'][None], m['W2'], m['b2'][None],
        m['W3'], m['b3'][None])
    return out
